# baseline (speedup 1.0000x reference)
import functools

import jax
import jax.numpy as jnp
from jax.experimental import pallas as pl
from jax.experimental.pallas import tpu as pltpu


BLK = 4096
N_SUB = 2


def _sub(xb, sigs, wf, bmat, n_tiles, n_classes):
    scores = jax.lax.dot_general(
        xb, sigs, (((1,), (1,)), ((), ())),
        preferred_element_type=jnp.float32)
    scores_t = scores.T
    iota_st = jax.lax.broadcasted_iota(jnp.int32, scores_t.shape, 0)
    m_t = jnp.max(scores_t, axis=0, keepdims=True)
    idx = jnp.min(jnp.where(scores_t == m_t, iota_st, n_tiles), axis=0)
    onehot_t = (iota_st == idx[None, :]).astype(jnp.float32)

    alll = jax.lax.dot_general(
        xb.astype(jnp.bfloat16), wf, (((1,), (1,)), ((), ())),
        preferred_element_type=jnp.float32)

    tvec = jax.lax.broadcasted_iota(
        jnp.int32, (n_tiles, 8), 0).astype(jnp.float32)
    idxf = jax.lax.dot_general(
        onehot_t, tvec, (((0,), (0,)), ((), ())),
        preferred_element_type=jnp.float32)[:, :1]

    s0 = alll[:, 0 * 128:1 * 128]
    s1 = alll[:, 1 * 128:2 * 128]
    s2 = alll[:, 2 * 128:3 * 128]
    s3 = alll[:, 3 * 128:4 * 128]
    m01 = jnp.where(idxf < 2.0, s0, s1)
    m23 = jnp.where(idxf < 6.0, s2, s3)
    u = jnp.where(idxf < 4.0, m01, m23)
    odd = idxf - 2.0 * jnp.floor(idxf * 0.5)
    logits = jnp.where(odd < 0.5, u[:, :n_classes], u[:, n_classes:])

    bsel = jax.lax.dot_general(
        onehot_t, bmat, (((0,), (0,)), ((), ())),
        preferred_element_type=jnp.float32)
    return logits + bsel, idx


def _body(x_ref, raw_ref, wf_ref, b_ref, out_ref, idx_ref, *, n_tiles, n_classes):
    rawv = raw_ref[:, :]
    sigs = jnp.where(rawv > 0.3, 1.0, jnp.where(rawv < -0.3, -1.0, 0.0))
    wf = wf_ref[:, :].astype(jnp.bfloat16)
    bmat = b_ref[:, :]
    sub = BLK // N_SUB
    for s in range(N_SUB):
        o, i = _sub(x_ref[s * sub:(s + 1) * sub, :], sigs, wf, bmat,
                    n_tiles, n_classes)
        out_ref[s * sub:(s + 1) * sub, :] = o
        idx_ref[s * sub:(s + 1) * sub] = i


@jax.jit
def kernel(x, raw, W, b):
    n_tok, d_model = x.shape
    n_tiles, n_classes, _ = W.shape
    wf = W.reshape(n_tiles * n_classes, d_model)
    grid = n_tok // BLK

    logits, idx3 = pl.pallas_call(
        functools.partial(_body, n_tiles=n_tiles, n_classes=n_classes),
        grid=(grid,),
        in_specs=[
            pl.BlockSpec((BLK, d_model), lambda i: (i, 0)),
            pl.BlockSpec((n_tiles, d_model), lambda i: (0, 0)),
            pl.BlockSpec((n_tiles * n_classes, d_model), lambda i: (0, 0)),
            pl.BlockSpec((n_tiles, n_classes), lambda i: (0, 0)),
        ],
        out_specs=[
            pl.BlockSpec((BLK, n_classes), lambda i: (i, 0)),
            pl.BlockSpec((BLK,), lambda i: (i,)),
        ],
        out_shape=[
            jax.ShapeDtypeStruct((n_tok, n_classes), jnp.float32),
            jax.ShapeDtypeStruct((n_tok,), jnp.int32),
        ],
        compiler_params=pltpu.CompilerParams(
            dimension_semantics=("parallel",)),
    )(x, raw, wf, b)

    return logits, idx3


# speedup vs baseline: 1.0559x; 1.0559x over previous
import functools

import jax
import jax.numpy as jnp
from jax.experimental import pallas as pl
from jax.experimental.pallas import tpu as pltpu


BLK = 4096
N_SUB = 4


def _sub(xb, sigs, wf, bmat, n_tiles, n_classes):
    scores = jax.lax.dot_general(
        xb, sigs, (((1,), (1,)), ((), ())),
        preferred_element_type=jnp.float32)
    scores_t = scores.T
    iota_st = jax.lax.broadcasted_iota(jnp.int32, scores_t.shape, 0)
    m_t = jnp.max(scores_t, axis=0, keepdims=True)
    idx = jnp.min(jnp.where(scores_t == m_t, iota_st, n_tiles), axis=0)
    onehot_t = (iota_st == idx[None, :]).astype(jnp.float32)

    alll = jax.lax.dot_general(
        xb.astype(jnp.bfloat16), wf, (((1,), (1,)), ((), ())),
        preferred_element_type=jnp.float32)

    tvec = jax.lax.broadcasted_iota(
        jnp.int32, (n_tiles, 8), 0).astype(jnp.float32)
    idxf = jax.lax.dot_general(
        onehot_t, tvec, (((0,), (0,)), ((), ())),
        preferred_element_type=jnp.float32)[:, :1]

    s0 = alll[:, 0 * 128:1 * 128]
    s1 = alll[:, 1 * 128:2 * 128]
    s2 = alll[:, 2 * 128:3 * 128]
    s3 = alll[:, 3 * 128:4 * 128]
    m01 = jnp.where(idxf < 2.0, s0, s1)
    m23 = jnp.where(idxf < 6.0, s2, s3)
    u = jnp.where(idxf < 4.0, m01, m23)
    odd = idxf - 2.0 * jnp.floor(idxf * 0.5)
    logits = jnp.where(odd < 0.5, u[:, :n_classes], u[:, n_classes:])

    bsel = jax.lax.dot_general(
        onehot_t, bmat, (((0,), (0,)), ((), ())),
        preferred_element_type=jnp.float32)
    return logits + bsel, idx


def _body(x_ref, raw_ref, wf_ref, b_ref, out_ref, idx_ref, *, n_tiles, n_classes):
    rawv = raw_ref[:, :]
    sigs = jnp.where(rawv > 0.3, 1.0, jnp.where(rawv < -0.3, -1.0, 0.0))
    wf = wf_ref[:, :].astype(jnp.bfloat16)
    bmat = b_ref[:, :]
    sub = BLK // N_SUB
    for s in range(N_SUB):
        o, i = _sub(x_ref[s * sub:(s + 1) * sub, :], sigs, wf, bmat,
                    n_tiles, n_classes)
        out_ref[s * sub:(s + 1) * sub, :] = o
        idx_ref[s * sub:(s + 1) * sub] = i


@jax.jit
def kernel(x, raw, W, b):
    n_tok, d_model = x.shape
    n_tiles, n_classes, _ = W.shape
    wf = W.reshape(n_tiles * n_classes, d_model)
    grid = n_tok // BLK

    logits, idx3 = pl.pallas_call(
        functools.partial(_body, n_tiles=n_tiles, n_classes=n_classes),
        grid=(grid,),
        in_specs=[
            pl.BlockSpec((BLK, d_model), lambda i: (i, 0)),
            pl.BlockSpec((n_tiles, d_model), lambda i: (0, 0)),
            pl.BlockSpec((n_tiles * n_classes, d_model), lambda i: (0, 0)),
            pl.BlockSpec((n_tiles, n_classes), lambda i: (0, 0)),
        ],
        out_specs=[
            pl.BlockSpec((BLK, n_classes), lambda i: (i, 0)),
            pl.BlockSpec((BLK,), lambda i: (i,)),
        ],
        out_shape=[
            jax.ShapeDtypeStruct((n_tok, n_classes), jnp.float32),
            jax.ShapeDtypeStruct((n_tok,), jnp.int32),
        ],
        compiler_params=pltpu.CompilerParams(
            dimension_semantics=("parallel",)),
    )(x, raw, wf, b)

    return logits, idx3
